# Initial kernel scaffold; baseline (speedup 1.0000x reference)
import jax
import jax.numpy as jnp
from jax.experimental import pallas as pl


def kernel(sr_data, tg_data, W_sr, W_tg):
    raise NotImplementedError("write your pallas kernel here")


# speedup vs baseline: 1.5154x; 1.5154x over previous
import functools

import jax
import jax.numpy as jnp
from jax import lax
from jax.experimental import pallas as pl
from jax.experimental.pallas import tpu as pltpu
from jax.experimental.pallas import tpu_sc as plsc

BATCH = 16384
EMBED_DIM = 128
CHUNK = 128

_info = plsc.get_sparse_core_info()
_NC, _NS = _info.num_cores, _info.num_subcores
_NW = _NC * _NS
_BPW = BATCH // _NW


def _body(sr_hbm, tg_hbm, wsr_hbm, wtg_hbm, out_sr, out_tg,
          idx_v, rows_v, sem):
    wid = lax.axis_index("s") * _NC + lax.axis_index("c")
    base = wid * _BPW
    n_chunks = _BPW // CHUNK

    for tbl_hbm, data_hbm, out_hbm in (
        (wsr_hbm, sr_hbm, out_sr),
        (wtg_hbm, tg_hbm, out_tg),
    ):
        pltpu.sync_copy(data_hbm.at[pl.ds(base, _BPW)], idx_v)
        copies = [
            pltpu.async_copy(
                tbl_hbm.at[idx_v.at[pl.ds(j * CHUNK, CHUNK)]],
                rows_v.at[pl.ds(j * CHUNK, CHUNK)],
                sem,
            )
            for j in range(n_chunks)
        ]
        for c in copies:
            c.wait()
        pltpu.sync_copy(rows_v, out_hbm.at[pl.ds(base, _BPW)])


def kernel(sr_data, tg_data, W_sr, W_tg):
    run = functools.partial(
        pl.kernel,
        mesh=plsc.VectorSubcoreMesh(core_axis_name="c", subcore_axis_name="s"),
        out_type=(
            jax.ShapeDtypeStruct((BATCH, EMBED_DIM), jnp.float32),
            jax.ShapeDtypeStruct((BATCH, EMBED_DIM), jnp.float32),
        ),
        scratch_types=[
            pltpu.VMEM((_BPW,), jnp.int32),
            pltpu.VMEM((_BPW, EMBED_DIM), jnp.float32),
            pltpu.SemaphoreType.DMA,
        ],
    )(_body)
    return run(sr_data.astype(jnp.int32), tg_data.astype(jnp.int32),
               W_sr, W_tg)


# speedup vs baseline: 1.5600x; 1.0294x over previous
import functools

import jax
import jax.numpy as jnp
from jax import lax
from jax.experimental import pallas as pl
from jax.experimental.pallas import tpu as pltpu
from jax.experimental.pallas import tpu_sc as plsc

BATCH = 16384
EMBED_DIM = 128
CHUNK = 128
NBUF = 7

_info = plsc.get_sparse_core_info()
_NC, _NS = _info.num_cores, _info.num_subcores
_NW = _NC * _NS
_BPW = BATCH // _NW
_NCH = 2 * _BPW // CHUNK
_CPT = _BPW // CHUNK


def _body(sr_hbm, tg_hbm, wsr_hbm, wtg_hbm, out_sr, out_tg,
          idx_v, bufs, gsem, wsem):
    wid = lax.axis_index("s") * _NC + lax.axis_index("c")
    base = wid * _BPW

    pltpu.sync_copy(sr_hbm.at[pl.ds(base, _BPW)], idx_v.at[pl.ds(0, _BPW)])
    pltpu.sync_copy(tg_hbm.at[pl.ds(base, _BPW)], idx_v.at[pl.ds(_BPW, _BPW)])

    def gather(c, b):
        tbl = wsr_hbm if c < _CPT else wtg_hbm
        return pltpu.async_copy(
            tbl.at[idx_v.at[pl.ds(c * CHUNK, CHUNK)]], bufs.at[b], gsem.at[b])

    def write(c, b):
        out = out_sr if c < _CPT else out_tg
        off = base + (c % _CPT) * CHUNK
        return pltpu.async_copy(bufs.at[b], out.at[pl.ds(off, CHUNK)],
                                wsem.at[b])

    g = [None] * NBUF
    w = [None] * NBUF
    for c in range(min(NBUF, _NCH)):
        g[c] = gather(c, c)
    for c in range(_NCH):
        b = c % NBUF
        g[b].wait()
        w[b] = write(c, b)
        nc = c + NBUF
        if nc < _NCH:
            w[b].wait()
            g[b] = gather(nc, b)
    for c in range(max(0, _NCH - NBUF), _NCH):
        w[c % NBUF].wait()


def kernel(sr_data, tg_data, W_sr, W_tg):
    run = functools.partial(
        pl.kernel,
        mesh=plsc.VectorSubcoreMesh(core_axis_name="c", subcore_axis_name="s"),
        out_type=(
            jax.ShapeDtypeStruct((BATCH, EMBED_DIM), jnp.float32),
            jax.ShapeDtypeStruct((BATCH, EMBED_DIM), jnp.float32),
        ),
        scratch_types=[
            pltpu.VMEM((2 * _BPW,), jnp.int32),
            pltpu.VMEM((NBUF, CHUNK, EMBED_DIM), jnp.float32),
            pltpu.SemaphoreType.DMA((NBUF,)),
            pltpu.SemaphoreType.DMA((NBUF,)),
        ],
    )(_body)
    return run(sr_data.astype(jnp.int32), tg_data.astype(jnp.int32),
               W_sr, W_tg)
